# baseline (speedup 1.0000x reference)
import jax
import jax.numpy as jnp
from jax import lax
from jax.experimental import pallas as pl
from jax.experimental.pallas import tpu as pltpu
from jax.experimental.pallas import tpu_sc as plsc

V = 1000000
H = V // 2
D = 64
B = 16384
NC = 2
NS = 16
NW = NC * NS
BPW = B // NW
HB = BPW // 2


def _nrsqrt(s):
    i = plsc.bitcast(s, jnp.int32)
    i = jnp.int32(0x5F3759DF) - lax.shift_right_logical(i, jnp.int32(1))
    y = plsc.bitcast(i, jnp.float32)
    for _ in range(3):
        y = y * (jnp.float32(1.5) - jnp.float32(0.5) * s * y * y)
    return y


def _gather_lo_body(w_hbm, idx_hbm, ia_hbm, rows_hbm, arow_hbm,
                    idx_v, ci_v, ia_v, ial_v, rows_v, a_v, sem, sem_a):
    wid = lax.axis_index("s") * NC + lax.axis_index("c")

    pltpu.sync_copy(idx_hbm.at[pl.ds(wid * 4, 4)], idx_v)
    pltpu.sync_copy(ia_hbm, ia_v)

    hmax = jnp.full((16,), H - 1, jnp.int32)
    for j in range(4):
        for k in range(8):
            v = idx_v[j, pl.ds(k * 16, 16)]
            ci_v[j, pl.ds(k * 16, 16)] = jnp.minimum(v, hmax)
    ial_v[pl.ds(0, 16)] = jnp.minimum(ia_v[pl.ds(0, 16)], hmax)

    handles = [pltpu.async_copy(w_hbm.at[ial_v], a_v, sem_a)]
    for j in range(4):
        handles.append(pltpu.async_copy(w_hbm.at[ci_v.at[j]],
                                        rows_v.at[pl.ds(j * 128, 128)], sem))
    for h in handles:
        h.wait()

    pltpu.sync_copy(rows_v, rows_hbm.at[pl.ds(wid * BPW, BPW)])

    @pl.when(wid == 0)
    def _():
        pltpu.sync_copy(a_v, arow_hbm)


def _gather_hi_body(w_hbm, idx_hbm, ia_hbm, rows_lo_hbm, arow_lo_hbm, out_hbm,
                    idx_v, ci_v, ia_v, iah_v, rows_v, rlo_v, a_v, alo_v,
                    out_v, sem, sem_a, sem_l):
    wid = lax.axis_index("s") * NC + lax.axis_index("c")

    pltpu.sync_copy(idx_hbm.at[pl.ds(wid * 4, 4)], idx_v)
    pltpu.sync_copy(ia_hbm, ia_v)
    pltpu.sync_copy(arow_lo_hbm, alo_v)

    zero16 = jnp.zeros((16,), jnp.int32)
    hsplat = jnp.full((16,), H, jnp.int32)
    hmax = jnp.full((16,), H - 1, jnp.int32)
    lanes = lax.iota(jnp.int32, 16)

    for j in range(4):
        for k in range(8):
            v = idx_v[j, pl.ds(k * 16, 16)]
            ci_v[j, pl.ds(k * 16, 16)] = jnp.minimum(
                jnp.maximum(v - hsplat, zero16), hmax)
    iav = ia_v[pl.ds(0, 16)]
    iah_v[pl.ds(0, 16)] = jnp.minimum(
        jnp.maximum(iav - hsplat, zero16), hmax)

    pltpu.async_copy(w_hbm.at[iah_v], a_v, sem_a).wait()
    amask = iav >= hsplat
    a_regs = [jnp.where(amask, a_v[0, pl.ds(k * 16, 16)],
                        alo_v[0, pl.ds(k * 16, 16)])
              for k in range(D // 16)]
    sa = jnp.float32(0)
    for k in range(D // 16):
        sq = a_regs[k] * a_regs[k]
        for l in range(16):
            sa = sa + sq[l]
    sa = jnp.maximum(sa, jnp.float32(1e-16))
    a_sc = [a_regs[k][l] for k in range(D // 16) for l in range(16)]

    for half in range(2):
        hl = [pltpu.async_copy(
                  rows_lo_hbm.at[pl.ds(wid * BPW + half * HB, HB)],
                  rlo_v, sem_l)]
        for j in range(2):
            c = half * 2 + j
            hl.append(pltpu.async_copy(
                w_hbm.at[ci_v.at[c]],
                rows_v.at[pl.ds(j * 128, 128)], sem))
        for h in hl:
            h.wait()

        def group(g, carry, half=half):
            rows16 = g * 16 + lanes
            gg = half * (HB // 16) + g
            v = plsc.load_gather(
                idx_v, [lax.shift_right_logical(gg, 3) + zero16,
                        jnp.bitwise_and(gg, 7) * 16 + lanes])
            hi = v >= hsplat
            acc_p = jnp.zeros((16,), jnp.float32)
            acc_q = jnp.zeros((16,), jnp.float32)
            for d in range(D):
                col = jnp.full((16,), d, jnp.int32)
                va = plsc.load_gather(rlo_v, [rows16, col])
                vb = plsc.load_gather(rows_v, [rows16, col])
                val = jnp.where(hi, vb, va)
                acc_p = acc_p + a_sc[d] * val
                acc_q = acc_q + val * val
            r = _nrsqrt(sa * jnp.maximum(acc_q, jnp.float32(1e-16)))
            out_v[pl.ds(half * HB + g * 16, 16)] = acc_p * r
            return carry

        lax.fori_loop(0, HB // 16, group, None)

    pltpu.sync_copy(out_v, out_hbm.at[pl.ds(wid * BPW, BPW)])


def kernel(x, weight):
    w_lo = weight[:H]
    w_hi = weight[H:]
    idx = x[:, 1].astype(jnp.int32).reshape(NW * 4, 128)
    ia16 = jnp.broadcast_to(x[0, 0].astype(jnp.int32)[None], (16,))

    mesh = plsc.VectorSubcoreMesh(core_axis_name="c", subcore_axis_name="s",
                                  num_cores=NC, num_subcores=NS)
    params = pltpu.CompilerParams(needs_layout_passes=False,
                                  use_tc_tiling_on_sc=False)

    rows_st, arow_st = pl.kernel(
        _gather_lo_body,
        out_type=(jax.ShapeDtypeStruct((B, D), jnp.float32),
                  jax.ShapeDtypeStruct((16, D), jnp.float32)),
        mesh=mesh,
        compiler_params=params,
        scratch_types=[
            pltpu.VMEM((4, 128), jnp.int32),
            pltpu.VMEM((4, 128), jnp.int32),
            pltpu.VMEM((16,), jnp.int32),
            pltpu.VMEM((16,), jnp.int32),
            pltpu.VMEM((BPW, D), jnp.float32),
            pltpu.VMEM((16, D), jnp.float32),
            pltpu.SemaphoreType.DMA,
            pltpu.SemaphoreType.DMA,
        ],
    )(w_lo, idx, ia16)

    return pl.kernel(
        _gather_hi_body,
        out_type=jax.ShapeDtypeStruct((B,), jnp.float32),
        mesh=mesh,
        compiler_params=params,
        scratch_types=[
            pltpu.VMEM((4, 128), jnp.int32),
            pltpu.VMEM((4, 128), jnp.int32),
            pltpu.VMEM((16,), jnp.int32),
            pltpu.VMEM((16,), jnp.int32),
            pltpu.VMEM((HB, D), jnp.float32),
            pltpu.VMEM((HB, D), jnp.float32),
            pltpu.VMEM((16, D), jnp.float32),
            pltpu.VMEM((16, D), jnp.float32),
            pltpu.VMEM((BPW,), jnp.float32),
            pltpu.SemaphoreType.DMA,
            pltpu.SemaphoreType.DMA,
            pltpu.SemaphoreType.DMA,
        ],
    )(w_hi, idx, ia16, rows_st, arow_st)


# speedup vs baseline: 296618.3418x; 296618.3418x over previous
import jax
import jax.numpy as jnp
from jax import lax
from jax.experimental import pallas as pl
from jax.experimental.pallas import tpu as pltpu
from jax.experimental.pallas import tpu_sc as plsc

D = 64
B = 16384
NC = 2
NS = 16
NW = NC * NS
BPW = B // NW
NGRP = BPW // 16
NSEM = 4


def _nrsqrt(s):
    i = plsc.bitcast(s, jnp.int32)
    i = jnp.int32(0x5F3759DF) - lax.shift_right_logical(i, jnp.int32(1))
    y = plsc.bitcast(i, jnp.float32)
    for _ in range(3):
        y = y * (jnp.float32(1.5) - jnp.float32(0.5) * s * y * y)
    return y


def _sc_body(weight_hbm, idx_hbm, ia_hbm, out_hbm,
             idx_v, ia_v, rows_v, a_v, out_v,
             sem0, sem1, sem2, sem3, sem_a):
    wid = lax.axis_index("s") * NC + lax.axis_index("c")
    sems = (sem0, sem1, sem2, sem3)

    pltpu.sync_copy(idx_hbm.at[pl.ds(wid * 4, 4)], idx_v)
    pltpu.sync_copy(ia_hbm, ia_v)

    ia = ia_v[pl.ds(0, 16)][0]
    handles = [pltpu.async_copy(weight_hbm.at[pl.ds(ia, 1)], a_v, sem_a)]
    for j in range(4):
        for k in range(8):
            v = idx_v[j, pl.ds(k * 16, 16)]
            for l in range(16):
                i = j * 128 + k * 16 + l
                handles.append(pltpu.async_copy(
                    weight_hbm.at[pl.ds(v[l], 1)],
                    rows_v.at[pl.ds(i, 1)], sems[i % NSEM]))
    for h in handles:
        h.wait()

    a_regs = [a_v[0, pl.ds(k * 16, 16)] for k in range(D // 16)]
    sa = jnp.float32(0)
    for k in range(D // 16):
        sq = a_regs[k] * a_regs[k]
        for l in range(16):
            sa = sa + sq[l]
    sa = jnp.maximum(sa, jnp.float32(1e-16))

    lanes = lax.iota(jnp.int32, 16)

    def group(g, carry):
        row_idx = g * 16 + lanes
        acc_dot = jnp.zeros((16,), jnp.float32)
        acc_sq = jnp.zeros((16,), jnp.float32)
        for d in range(D):
            col = jnp.full((16,), d, jnp.int32)
            vals = plsc.load_gather(rows_v, [row_idx, col])
            a_d = a_regs[d // 16][d % 16]
            acc_dot = acc_dot + a_d * vals
            acc_sq = acc_sq + vals * vals
        r = _nrsqrt(sa * jnp.maximum(acc_sq, jnp.float32(1e-16)))
        out_v[pl.ds(g * 16, 16)] = acc_dot * r
        return carry

    lax.fori_loop(0, NGRP, group, None)

    pltpu.sync_copy(out_v, out_hbm.at[pl.ds(wid * BPW, BPW)])


def kernel(x, weight):
    idx = x[:, 1].astype(jnp.int32).reshape(NW * 4, 128)
    ia = jnp.broadcast_to(x[0, 0].astype(jnp.int32)[None], (16,))
    run = pl.kernel(
        _sc_body,
        out_type=jax.ShapeDtypeStruct((B,), jnp.float32),
        mesh=plsc.VectorSubcoreMesh(core_axis_name="c", subcore_axis_name="s",
                                    num_cores=NC, num_subcores=NS),
        compiler_params=pltpu.CompilerParams(needs_layout_passes=False),
        scratch_types=[
            pltpu.VMEM((4, 128), jnp.int32),
            pltpu.VMEM((16,), jnp.int32),
            pltpu.VMEM((BPW, D), jnp.float32),
            pltpu.VMEM((1, D), jnp.float32),
            pltpu.VMEM((BPW,), jnp.float32),
            pltpu.SemaphoreType.DMA,
            pltpu.SemaphoreType.DMA,
            pltpu.SemaphoreType.DMA,
            pltpu.SemaphoreType.DMA,
            pltpu.SemaphoreType.DMA,
        ],
    )(weight, idx, ia)
